# baseline (speedup 1.0000x reference)
import jax
import jax.numpy as jnp
from jax import lax
from jax.experimental import pallas as pl
from jax.experimental.pallas import tpu as pltpu
from jax.experimental.pallas import tpu_sc as plsc

N_USER = 50000
N_ITEM = 50000
N = N_USER + N_ITEM
D = 32
E = 1600000
N_LAYERS = 3
B = 4096

NC = 2
NS = 16
L = 16

KB = 512
SUB = 128
NSUB = KB // SUB
BLKS = 198
EPT = BLKS * KB
EPAD = EPT * NS
RPT = 6272
NP = RPT * NS

_GATHER_DN = lax.GatherDimensionNumbers(
    offset_dims=(), collapsed_slice_dims=(0,), start_index_map=(0,))


def _bcast_lane(vals, i):
    return lax.gather(vals, jnp.full((L, 1), i, jnp.int32),
                      dimension_numbers=_GATHER_DN, slice_sizes=(1,),
                      mode=lax.GatherScatterMode.PROMISE_IN_BOUNDS)


def _sc_body(emb0, col2d, row2d, val1d, users2d, items2d,
             up_out, ip_out, lyr1, lyr2, lyr3,
             colb0, rowb0, valb0, colb1, rowb1, valb1, colb2, rowb2, valb2,
             grows0, grows1, acc, gsem, lsem, ssem):
    sc = lax.axis_index("c")
    t = lax.axis_index("s")

    zero16 = jnp.zeros((L,), jnp.float32)
    zero16i = jnp.zeros((L,), jnp.int32)
    r0 = t * RPT

    lin = [(colb0, rowb0, valb0), (colb1, rowb1, valb1), (colb2, rowb2, valb2)]
    grows = [grows0, grows1]

    def fire_linear(li_set, blk):
        colb, rowb, valb = lin[li_set]
        pltpu.async_copy(col2d.at[pl.ds(blk * NSUB, NSUB)], colb, lsem)
        pltpu.async_copy(row2d.at[pl.ds(blk * NSUB, NSUB)], rowb, lsem)
        pltpu.async_copy(val1d.at[pl.ds(blk * KB, KB)], valb, lsem)

    def wait_linear(li_set):
        colb, rowb, valb = lin[li_set]
        pltpu.make_async_copy(col2d.at[pl.ds(0, NSUB)], colb, lsem).wait()
        pltpu.make_async_copy(row2d.at[pl.ds(0, NSUB)], rowb, lsem).wait()
        pltpu.make_async_copy(val1d.at[pl.ds(0, KB)], valb, lsem).wait()

    def fire_gathers(src, li_set, gset):
        colb = lin[li_set][0]
        g = grows[gset]
        for j in range(NSUB):
            pltpu.async_copy(src.at[colb.at[j]],
                             g.at[pl.ds(j * SUB, SUB)], gsem)

    def wait_gathers(src, gset):
        g = grows[gset]
        for j in range(NSUB):
            pltpu.make_async_copy(src.at[pl.ds(0, SUB)],
                                  g.at[pl.ds(j * SUB, SUB)], gsem).wait()

    def fire_scatters(li_set, gset):
        rowb = lin[li_set][1]
        g = grows[gset]
        for j in range(NSUB):
            pltpu.async_copy(g.at[pl.ds(j * SUB, SUB)],
                             acc.at[rowb.at[j]], ssem, add=True)

    def drain_scatters(gset):
        g = grows[gset]
        for j in range(NSUB):
            pltpu.make_async_copy(g.at[pl.ds(j * SUB, SUB)],
                                  acc.at[pl.ds(0, SUB)], ssem).wait()

    def compute(li_set, gset):
        valb = lin[li_set][2]
        g = grows[gset]

        def grp_body(gi, _):
            vals = valb[pl.ds(gi * L, L)]
            for i in range(L):
                e = gi * L + i
                g[e] = g[e] * _bcast_lane(vals, i)
            return 0
        lax.fori_loop(0, KB // L, grp_body, 0)

    lyrs = [lyr1, lyr2, lyr3]
    for li in range(N_LAYERS):
        src = (emb0 if li == 0 else lyrs[li - 1]).at[sc]
        tb = t * BLKS

        def zr_body(i, _):
            grows0[i] = zero16
            grows1[i] = zero16
            return 0
        lax.fori_loop(0, KB, zr_body, 0)
        for k in range(RPT // KB):
            pltpu.sync_copy(grows0, acc.at[pl.ds(r0 + k * KB, KB)])
        pltpu.sync_copy(grows0.at[pl.ds(0, RPT - (RPT // KB) * KB)],
                        acc.at[pl.ds(r0 + (RPT // KB) * KB,
                                     RPT - (RPT // KB) * KB)])
        for i in range(NSUB):
            for k in range(SUB // L):
                rowb2[i, pl.ds(k * L, L)] = zero16i
        for j in range(NSUB):
            pltpu.async_copy(grows1.at[pl.ds(j * SUB, SUB)],
                             acc.at[rowb2.at[j]], ssem, add=True)
        fire_linear(0, tb)
        wait_linear(0)
        fire_gathers(src, 0, 0)
        fire_linear(1, tb + 1)
        plsc.subcore_barrier()

        def six_body(k, _, src=src):
            x0 = 6 * k
            for u in range(6):
                x = x0 + u
                lu, gu = u % 3, u % 2
                wait_gathers(src, gu)
                drain_scatters((u + 1) % 2)
                wait_linear((u + 1) % 3)
                fire_gathers(src, (u + 1) % 3, (u + 1) % 2)
                compute(lu, gu)
                fire_scatters(lu, gu)
                fire_linear((u + 2) % 3,
                            tb + jnp.minimum(x + 2, BLKS - 1))
            return 0

        lax.fori_loop(0, BLKS // 6, six_body, 0)
        wait_gathers(src, 0)
        drain_scatters(1)
        wait_linear(1)
        plsc.subcore_barrier()

        dst = lyrs[li].at[sc]
        pltpu.sync_copy(acc.at[pl.ds(r0, RPT)], dst.at[pl.ds(r0, RPT)])

    srcs = [emb0.at[sc]] + [ly.at[sc] for ly in lyrs]
    for idx2d, outp in ((users2d, up_out), (items2d, ip_out)):
        pltpu.sync_copy(idx2d.at[pl.ds(2 * t, 2)], colb0.at[pl.ds(0, 2)])
        for j in range(2):
            descs = [
                pltpu.async_copy(s.at[colb0.at[j]],
                                 grows0.at[pl.ds(k * SUB, SUB)], gsem)
                for k, s in enumerate(srcs)
            ]
            for d in descs:
                d.wait()

            def avg_body(i, _):
                grows0[i] = (grows0[i] + grows0[SUB + i] + grows0[2 * SUB + i]
                             + grows0[3 * SUB + i]) * 0.25
                return 0
            lax.fori_loop(0, SUB, avg_body, 0)
            pltpu.sync_copy(grows0.at[pl.ds(0, SUB)],
                            outp.at[sc].at[pl.ds(t * 256 + j * SUB, SUB)])


def _propagate(emb0, col2d, row2d, val1d, users2d, items2d):
    mesh = plsc.VectorSubcoreMesh(core_axis_name="c", subcore_axis_name="s")
    f32 = jnp.float32
    i32 = jnp.int32
    lin_set = (pltpu.VMEM((NSUB, SUB), i32),
               pltpu.VMEM((NSUB, SUB), i32),
               pltpu.VMEM((KB,), f32))
    kfn = pl.kernel(
        _sc_body,
        out_type=(
            jax.ShapeDtypeStruct((NC, B, L), f32),
            jax.ShapeDtypeStruct((NC, B, L), f32),
            jax.ShapeDtypeStruct((NC, NP, L), f32),
            jax.ShapeDtypeStruct((NC, NP, L), f32),
            jax.ShapeDtypeStruct((NC, NP, L), f32),
        ),
        mesh=mesh,
        compiler_params=pltpu.CompilerParams(use_tc_tiling_on_sc=False),
        scratch_types=lin_set * 3 + (
            pltpu.VMEM((KB, L), f32),
            pltpu.VMEM((KB, L), f32),
            pltpu.VMEM_SHARED((NP, L), f32),
            pltpu.SemaphoreType.DMA,
            pltpu.SemaphoreType.DMA,
            pltpu.SemaphoreType.DMA,
        ),
    )
    return kfn(emb0, col2d, row2d, val1d, users2d, items2d)


def _mm_body(u_ref, it_ref, o_ref):
    prod = lax.dot_general(u_ref[...], it_ref[...],
                           (((1,), (1,)), ((), ())),
                           preferred_element_type=jnp.float32)
    o_ref[...] = jax.nn.sigmoid(prod)


def _score(users_emb, items_emb):
    BM = 512
    grid = (B // BM, B // BM)
    return pl.pallas_call(
        _mm_body,
        grid=grid,
        in_specs=[
            pl.BlockSpec((BM, D), lambda i, j: (i, 0)),
            pl.BlockSpec((BM, D), lambda i, j: (j, 0)),
        ],
        out_specs=pl.BlockSpec((BM, BM), lambda i, j: (i, j)),
        out_shape=jax.ShapeDtypeStruct((B, B), jnp.float32),
    )(users_emb, items_emb)


def kernel(users, items, user_emb, item_emb, adj_row, adj_col, adj_val):
    all0 = jnp.concatenate([user_emb, item_emb], axis=0)
    all0 = jnp.pad(all0, ((0, NP - N), (0, 0)))
    emb0 = jnp.stack([all0[:, :L], all0[:, L:]])

    pad = EPAD - E
    col2d = jnp.pad(adj_col, (0, pad)).reshape(EPAD // SUB, SUB)
    row2d = jnp.pad(adj_row, (0, pad)).reshape(EPAD // SUB, SUB)
    val1d = jnp.pad(adj_val, (0, pad))
    users2d = users.reshape(B // SUB, SUB)
    items2d = (items + N_USER).reshape(B // SUB, SUB)

    up, ip, _, _, _ = _propagate(emb0, col2d, row2d, val1d, users2d, items2d)
    users_emb = up.transpose(1, 0, 2).reshape(B, D)
    items_emb = ip.transpose(1, 0, 2).reshape(B, D)
    return _score(users_emb, items_emb)


# speedup vs baseline: 1.0151x; 1.0151x over previous
import jax
import jax.numpy as jnp
from jax import lax
from jax.experimental import pallas as pl
from jax.experimental.pallas import tpu as pltpu
from jax.experimental.pallas import tpu_sc as plsc

N_USER = 50000
N_ITEM = 50000
N = N_USER + N_ITEM
D = 32
E = 1600000
N_LAYERS = 3
B = 4096

NC = 2
NS = 16
L = 16

KB = 384
SUB = 128
NSUB = KB // SUB
EBR = 2 * NSUB
BLKS = 264
EPT = BLKS * KB
EPAD = EPT * NS
RPT = 6272
NP = RPT * NS

_GATHER_DN = lax.GatherDimensionNumbers(
    offset_dims=(), collapsed_slice_dims=(0,), start_index_map=(0,))


def _bcast_lane(vals, i):
    return lax.gather(vals, jnp.full((L, 1), i, jnp.int32),
                      dimension_numbers=_GATHER_DN, slice_sizes=(1,),
                      mode=lax.GatherScatterMode.PROMISE_IN_BOUNDS)


def _sc_body(emb0, ebh, val1d, users2d, items2d,
             up_out, ip_out, lyr1, lyr2, lyr3,
             eb0, eb1, eb2, eb3, eb4, eb5,
             vb0, vb1, vb2, vb3, vb4, vb5,
             grows0, grows1, grows2, acc,
             gsem0, gsem1, gsem2, lsem, ssem):
    sc = lax.axis_index("c")
    t = lax.axis_index("s")

    zero16 = jnp.zeros((L,), jnp.float32)
    zero16i = jnp.zeros((L,), jnp.int32)
    r0 = t * RPT

    ebs = [(eb0, vb0), (eb1, vb1), (eb2, vb2), (eb3, vb3), (eb4, vb4),
           (eb5, vb5)]
    grows = [grows0, grows1, grows2]
    gsems = [gsem0, gsem1, gsem2]

    def fire_linear(ebv, blk):
        eb, vb = ebv
        pltpu.async_copy(ebh.at[blk], eb, lsem)
        pltpu.async_copy(val1d.at[pl.ds(blk * KB, KB)], vb, lsem)

    def wait_linear(ebv):
        eb, vb = ebv
        pltpu.make_async_copy(ebh.at[0], eb, lsem).wait()
        pltpu.make_async_copy(val1d.at[pl.ds(0, KB)], vb, lsem).wait()

    def fire_gathers(src, ebv, gi):
        eb = ebv[0]
        g, gs = grows[gi], gsems[gi]
        for j in range(NSUB):
            pltpu.async_copy(src.at[eb.at[j]],
                             g.at[pl.ds(j * SUB, SUB)], gs)

    def wait_gathers(src, gi):
        g, gs = grows[gi], gsems[gi]
        for j in range(NSUB):
            pltpu.make_async_copy(src.at[pl.ds(0, SUB)],
                                  g.at[pl.ds(j * SUB, SUB)], gs).wait()

    def fire_scatters(ebv, gi):
        eb = ebv[0]
        g = grows[gi]
        for j in range(NSUB):
            pltpu.async_copy(g.at[pl.ds(j * SUB, SUB)],
                             acc.at[eb.at[NSUB + j]], ssem, add=True)

    def drain_scatters():
        for j in range(NSUB):
            pltpu.make_async_copy(grows0.at[pl.ds(j * SUB, SUB)],
                                  acc.at[pl.ds(0, SUB)], ssem).wait()

    def compute(ebv, gi):
        vb = ebv[1]
        g = grows[gi]

        def grp_body(gg, _):
            vals = vb[pl.ds(gg * L, L)]
            for i in range(L):
                e = gg * L + i
                g[e] = g[e] * _bcast_lane(vals, i)
            return 0
        lax.fori_loop(0, KB // L, grp_body, 0)

    lyrs = [lyr1, lyr2, lyr3]
    for li in range(N_LAYERS):
        src = (emb0 if li == 0 else lyrs[li - 1]).at[sc]
        tb = t * BLKS

        def zr_body(i, _):
            grows0[i] = zero16
            grows2[i] = zero16
            return 0
        lax.fori_loop(0, KB, zr_body, 0)
        for k in range(RPT // KB):
            pltpu.sync_copy(grows0, acc.at[pl.ds(r0 + k * KB, KB)])
        pltpu.sync_copy(grows0.at[pl.ds(0, RPT - (RPT // KB) * KB)],
                        acc.at[pl.ds(r0 + (RPT // KB) * KB,
                                     RPT - (RPT // KB) * KB)])
        for i in range(NSUB):
            for k in range(SUB // L):
                eb5[NSUB + i, pl.ds(k * L, L)] = zero16i
        for j in range(NSUB):
            pltpu.async_copy(grows2.at[pl.ds(j * SUB, SUB)],
                             acc.at[eb5.at[NSUB + j]], ssem, add=True)
        fire_linear(ebs[0], tb)
        wait_linear(ebs[0])
        fire_gathers(src, ebs[0], 0)
        fire_linear(ebs[1], tb + 1)
        wait_linear(ebs[1])
        fire_gathers(src, ebs[1], 1)
        fire_linear(ebs[2], tb + 2)
        plsc.subcore_barrier()

        def six_body(k, _, src=src):
            x0 = 6 * k
            for u in range(6):
                x = x0 + u
                wait_gathers(src, u % 3)
                drain_scatters()
                wait_linear(ebs[(u + 2) % 6])
                fire_gathers(src, ebs[(u + 2) % 6], (u + 2) % 3)
                compute(ebs[u], u % 3)
                fire_scatters(ebs[u], u % 3)
                fire_linear(ebs[(u + 3) % 6],
                            tb + jnp.minimum(x + 3, BLKS - 1))
            return 0

        lax.fori_loop(0, BLKS // 6, six_body, 0)
        wait_gathers(src, 0)
        wait_gathers(src, 1)
        drain_scatters()
        wait_linear(ebs[2])
        plsc.subcore_barrier()

        dst = lyrs[li].at[sc]
        pltpu.sync_copy(acc.at[pl.ds(r0, RPT)], dst.at[pl.ds(r0, RPT)])

    srcs = [emb0.at[sc]] + [ly.at[sc] for ly in lyrs]
    for idx2d, outp in ((users2d, up_out), (items2d, ip_out)):
        pltpu.sync_copy(idx2d.at[pl.ds(2 * t, 2)], eb0.at[pl.ds(0, 2)])
        for j in range(2):
            descs = []
            for k, s in enumerate(srcs):
                g = grows0 if k < 3 else grows1
                descs.append(pltpu.async_copy(
                    s.at[eb0.at[j]],
                    g.at[pl.ds((k % 3) * SUB, SUB)], gsem0))
            for d in descs:
                d.wait()

            def avg_body(i, _):
                grows0[i] = (grows0[i] + grows0[SUB + i]
                             + grows0[2 * SUB + i] + grows1[i]) * 0.25
                return 0
            lax.fori_loop(0, SUB, avg_body, 0)
            pltpu.sync_copy(grows0.at[pl.ds(0, SUB)],
                            outp.at[sc].at[pl.ds(t * 256 + j * SUB, SUB)])


def _propagate(emb0, ebh, val1d, users2d, items2d):
    mesh = plsc.VectorSubcoreMesh(core_axis_name="c", subcore_axis_name="s")
    f32 = jnp.float32
    i32 = jnp.int32
    kfn = pl.kernel(
        _sc_body,
        out_type=(
            jax.ShapeDtypeStruct((NC, B, L), f32),
            jax.ShapeDtypeStruct((NC, B, L), f32),
            jax.ShapeDtypeStruct((NC, NP, L), f32),
            jax.ShapeDtypeStruct((NC, NP, L), f32),
            jax.ShapeDtypeStruct((NC, NP, L), f32),
        ),
        mesh=mesh,
        compiler_params=pltpu.CompilerParams(use_tc_tiling_on_sc=False),
        scratch_types=(pltpu.VMEM((EBR, SUB), i32),) * 6
        + (pltpu.VMEM((KB,), f32),) * 6 + (
            pltpu.VMEM((KB, L), f32),
            pltpu.VMEM((KB, L), f32),
            pltpu.VMEM((KB, L), f32),
            pltpu.VMEM_SHARED((NP, L), f32),
            pltpu.SemaphoreType.DMA,
            pltpu.SemaphoreType.DMA,
            pltpu.SemaphoreType.DMA,
            pltpu.SemaphoreType.DMA,
            pltpu.SemaphoreType.DMA,
        ),
    )
    return kfn(emb0, ebh, val1d, users2d, items2d)


def _mm_body(u_ref, it_ref, o_ref):
    prod = lax.dot_general(u_ref[...], it_ref[...],
                           (((1,), (1,)), ((), ())),
                           preferred_element_type=jnp.float32)
    o_ref[...] = jax.nn.sigmoid(prod)


def _score(users_emb, items_emb):
    BM = 512
    grid = (B // BM, B // BM)
    return pl.pallas_call(
        _mm_body,
        grid=grid,
        in_specs=[
            pl.BlockSpec((BM, D), lambda i, j: (i, 0)),
            pl.BlockSpec((BM, D), lambda i, j: (j, 0)),
        ],
        out_specs=pl.BlockSpec((BM, BM), lambda i, j: (i, j)),
        out_shape=jax.ShapeDtypeStruct((B, B), jnp.float32),
    )(users_emb, items_emb)


def kernel(users, items, user_emb, item_emb, adj_row, adj_col, adj_val):
    all0 = jnp.concatenate([user_emb, item_emb], axis=0)
    all0 = jnp.pad(all0, ((0, NP - N), (0, 0)))
    emb0 = jnp.stack([all0[:, :L], all0[:, L:]])

    pad = EPAD - E
    nblk = EPAD // KB
    col3 = jnp.pad(adj_col, (0, pad)).reshape(nblk, NSUB, SUB)
    row3 = jnp.pad(adj_row, (0, pad)).reshape(nblk, NSUB, SUB)
    val1d = jnp.pad(adj_val, (0, pad))
    ebh = jnp.concatenate([col3, row3], axis=1)
    users2d = users.reshape(B // SUB, SUB)
    items2d = (items + N_USER).reshape(B // SUB, SUB)

    up, ip, _, _, _ = _propagate(emb0, ebh, val1d, users2d, items2d)
    users_emb = up.transpose(1, 0, 2).reshape(B, D)
    items_emb = ip.transpose(1, 0, 2).reshape(B, D)
    return _score(users_emb, items_emb)


# speedup vs baseline: 1.0151x; 1.0000x over previous
import jax
import jax.numpy as jnp
from jax import lax
from jax.experimental import pallas as pl
from jax.experimental.pallas import tpu as pltpu
from jax.experimental.pallas import tpu_sc as plsc

N_USER = 50000
N_ITEM = 50000
N = N_USER + N_ITEM
D = 32
E = 1600000
N_LAYERS = 3
B = 4096

NC = 2
NS = 16
L = 16

KB = 384
SUB = 128
NSUB = KB // SUB
EBR = 2 * NSUB
BLKS = 264
EPT = BLKS * KB
EPAD = EPT * NS
RPT = 6272
NP = RPT * NS

_GATHER_DN = lax.GatherDimensionNumbers(
    offset_dims=(), collapsed_slice_dims=(0,), start_index_map=(0,))


def _bcast_lane(vals, i):
    return lax.gather(vals, jnp.full((L, 1), i, jnp.int32),
                      dimension_numbers=_GATHER_DN, slice_sizes=(1,),
                      mode=lax.GatherScatterMode.PROMISE_IN_BOUNDS)


def _sc_body(emb0, ebh, val1d, users2d, items2d,
             up_out, ip_out, lyr1, lyr2, lyr3,
             eb0, eb1, eb2, eb3, eb4, eb5,
             vb0, vb1, vb2, vb3, vb4, vb5,
             grows0, grows1, grows2, acc,
             gsem0, gsem1, gsem2, lsem, ssem):
    sc = lax.axis_index("c")
    t = lax.axis_index("s")

    zero16 = jnp.zeros((L,), jnp.float32)
    zero16i = jnp.zeros((L,), jnp.int32)
    r0 = t * RPT

    ebs = [(eb0, vb0), (eb1, vb1), (eb2, vb2), (eb3, vb3), (eb4, vb4),
           (eb5, vb5)]
    grows = [grows0, grows1, grows2]
    gsems = [gsem0, gsem1, gsem2]

    def fire_linear(ebv, blk):
        eb, vb = ebv
        pltpu.async_copy(ebh.at[pl.ds(blk * EBR, EBR)], eb, lsem)
        pltpu.async_copy(val1d.at[pl.ds(blk * KB, KB)], vb, lsem)

    def wait_linear(ebv):
        eb, vb = ebv
        pltpu.make_async_copy(ebh.at[pl.ds(0, EBR)], eb, lsem).wait()
        pltpu.make_async_copy(val1d.at[pl.ds(0, KB)], vb, lsem).wait()

    def fire_gathers(src, ebv, gi):
        eb = ebv[0]
        g, gs = grows[gi], gsems[gi]
        for j in range(NSUB):
            pltpu.async_copy(src.at[eb.at[j]],
                             g.at[pl.ds(j * SUB, SUB)], gs)

    def wait_gathers(src, gi):
        g, gs = grows[gi], gsems[gi]
        for j in range(NSUB):
            pltpu.make_async_copy(src.at[pl.ds(0, SUB)],
                                  g.at[pl.ds(j * SUB, SUB)], gs).wait()

    def fire_scatters(ebv, gi):
        eb = ebv[0]
        g = grows[gi]
        for j in range(NSUB):
            pltpu.async_copy(g.at[pl.ds(j * SUB, SUB)],
                             acc.at[eb.at[NSUB + j]], ssem, add=True)

    def drain_scatters():
        for j in range(NSUB):
            pltpu.make_async_copy(grows0.at[pl.ds(j * SUB, SUB)],
                                  acc.at[pl.ds(0, SUB)], ssem).wait()

    def compute(ebv, gi):
        vb = ebv[1]
        g = grows[gi]

        def grp_body(gg, _):
            vals = vb[pl.ds(gg * L, L)]
            for i in range(L):
                e = gg * L + i
                g[e] = g[e] * _bcast_lane(vals, i)
            return 0
        lax.fori_loop(0, KB // L, grp_body, 0)

    lyrs = [lyr1, lyr2, lyr3]
    for li in range(N_LAYERS):
        src = (emb0 if li == 0 else lyrs[li - 1]).at[sc]
        tb = t * BLKS

        def zr_body(i, _):
            grows0[i] = zero16
            grows2[i] = zero16
            return 0
        lax.fori_loop(0, KB, zr_body, 0)
        for k in range(RPT // KB):
            pltpu.sync_copy(grows0, acc.at[pl.ds(r0 + k * KB, KB)])
        pltpu.sync_copy(grows0.at[pl.ds(0, RPT - (RPT // KB) * KB)],
                        acc.at[pl.ds(r0 + (RPT // KB) * KB,
                                     RPT - (RPT // KB) * KB)])
        for i in range(NSUB):
            for k in range(SUB // L):
                eb5[NSUB + i, pl.ds(k * L, L)] = zero16i
        for j in range(NSUB):
            pltpu.async_copy(grows2.at[pl.ds(j * SUB, SUB)],
                             acc.at[eb5.at[NSUB + j]], ssem, add=True)
        fire_linear(ebs[0], tb)
        wait_linear(ebs[0])
        fire_gathers(src, ebs[0], 0)
        fire_linear(ebs[1], tb + 1)
        wait_linear(ebs[1])
        fire_gathers(src, ebs[1], 1)
        fire_linear(ebs[2], tb + 2)
        plsc.subcore_barrier()

        def six_body(k, _, src=src):
            x0 = 6 * k
            for u in range(6):
                x = x0 + u
                wait_gathers(src, u % 3)
                drain_scatters()
                wait_linear(ebs[(u + 2) % 6])
                fire_gathers(src, ebs[(u + 2) % 6], (u + 2) % 3)
                compute(ebs[u], u % 3)
                fire_scatters(ebs[u], u % 3)
                fire_linear(ebs[(u + 3) % 6],
                            tb + jnp.minimum(x + 3, BLKS - 1))
            return 0

        lax.fori_loop(0, BLKS // 6, six_body, 0)
        wait_gathers(src, 0)
        wait_gathers(src, 1)
        drain_scatters()
        wait_linear(ebs[2])
        plsc.subcore_barrier()

        dst = lyrs[li].at[sc]
        pltpu.sync_copy(acc.at[pl.ds(r0, RPT)], dst.at[pl.ds(r0, RPT)])

    srcs = [emb0.at[sc]] + [ly.at[sc] for ly in lyrs]
    for idx2d, outp in ((users2d, up_out), (items2d, ip_out)):
        pltpu.sync_copy(idx2d.at[pl.ds(2 * t, 2)], eb0.at[pl.ds(0, 2)])
        for j in range(2):
            descs = []
            for k, s in enumerate(srcs):
                g = grows0 if k < 3 else grows1
                descs.append(pltpu.async_copy(
                    s.at[eb0.at[j]],
                    g.at[pl.ds((k % 3) * SUB, SUB)], gsem0))
            for d in descs:
                d.wait()

            def avg_body(i, _):
                grows0[i] = (grows0[i] + grows0[SUB + i]
                             + grows0[2 * SUB + i] + grows1[i]) * 0.25
                return 0
            lax.fori_loop(0, SUB, avg_body, 0)
            pltpu.sync_copy(grows0.at[pl.ds(0, SUB)],
                            outp.at[sc].at[pl.ds(t * 256 + j * SUB, SUB)])


def _propagate(emb0, ebh, val1d, users2d, items2d):
    mesh = plsc.VectorSubcoreMesh(core_axis_name="c", subcore_axis_name="s")
    f32 = jnp.float32
    i32 = jnp.int32
    kfn = pl.kernel(
        _sc_body,
        out_type=(
            jax.ShapeDtypeStruct((NC, B, L), f32),
            jax.ShapeDtypeStruct((NC, B, L), f32),
            jax.ShapeDtypeStruct((NC, NP, L), f32),
            jax.ShapeDtypeStruct((NC, NP, L), f32),
            jax.ShapeDtypeStruct((NC, NP, L), f32),
        ),
        mesh=mesh,
        compiler_params=pltpu.CompilerParams(use_tc_tiling_on_sc=False),
        scratch_types=(pltpu.VMEM((EBR, SUB), i32),) * 6
        + (pltpu.VMEM((KB,), f32),) * 6 + (
            pltpu.VMEM((KB, L), f32),
            pltpu.VMEM((KB, L), f32),
            pltpu.VMEM((KB, L), f32),
            pltpu.VMEM_SHARED((NP, L), f32),
            pltpu.SemaphoreType.DMA,
            pltpu.SemaphoreType.DMA,
            pltpu.SemaphoreType.DMA,
            pltpu.SemaphoreType.DMA,
            pltpu.SemaphoreType.DMA,
        ),
    )
    return kfn(emb0, ebh, val1d, users2d, items2d)


def _mm_body(u_ref, it_ref, o_ref):
    prod = lax.dot_general(u_ref[...], it_ref[...],
                           (((1,), (1,)), ((), ())),
                           preferred_element_type=jnp.float32)
    o_ref[...] = jax.nn.sigmoid(prod)


def _score(users_emb, items_emb):
    BM = 512
    grid = (B // BM, B // BM)
    return pl.pallas_call(
        _mm_body,
        grid=grid,
        in_specs=[
            pl.BlockSpec((BM, D), lambda i, j: (i, 0)),
            pl.BlockSpec((BM, D), lambda i, j: (j, 0)),
        ],
        out_specs=pl.BlockSpec((BM, BM), lambda i, j: (i, j)),
        out_shape=jax.ShapeDtypeStruct((B, B), jnp.float32),
    )(users_emb, items_emb)


def kernel(users, items, user_emb, item_emb, adj_row, adj_col, adj_val):
    all0 = jnp.concatenate([user_emb, item_emb], axis=0)
    all0 = jnp.pad(all0, ((0, NP - N), (0, 0)))
    emb0 = jnp.stack([all0[:, :L], all0[:, L:]])

    pad = EPAD - E
    nblk = EPAD // KB
    col3 = jnp.pad(adj_col, (0, pad)).reshape(nblk, NSUB, SUB)
    row3 = jnp.pad(adj_row, (0, pad)).reshape(nblk, NSUB, SUB)
    val1d = jnp.pad(adj_val, (0, pad))
    ebh = jnp.concatenate([col3, row3], axis=1).reshape(
        nblk * EBR, SUB)
    users2d = users.reshape(B // SUB, SUB)
    items2d = (items + N_USER).reshape(B // SUB, SUB)

    up, ip, _, _, _ = _propagate(emb0, ebh, val1d, users2d, items2d)
    users_emb = up.transpose(1, 0, 2).reshape(B, D)
    items_emb = ip.transpose(1, 0, 2).reshape(B, D)
    return _score(users_emb, items_emb)
